# baseline (speedup 1.0000x reference)
import functools

import jax
import jax.numpy as jnp
from jax import lax
from jax.experimental import pallas as pl
from jax.experimental.pallas import tpu as pltpu
from jax.experimental.pallas import tpu_sc as plsc

N_NODES = 50000
HIDDEN = 64
SKEL = 256
FEAT = 16

NPAD = 51200
PROWS = NPAD * FEAT // 128
N_EDGES = 800000
EPAD = 819200
SLEN = 512
NSTR = 2
ECHUNKS = EPAD // (SLEN * NSTR)
TILE_OUT = NPAD // 16


def _sc_segsum(table_ref, src_ref, dst_ref, zeros_ref, out_ref,
               ebs, ebd, rowsv, acc, lsem, gsem, ssem,
               *, n_phases, split_edges):
    c = lax.axis_index("c")
    s = lax.axis_index("s")
    zbase = s * TILE_OUT

    if split_edges:
        n0 = (ECHUNKS // 2) // 16 - 9
        n1 = (ECHUNKS // 2) // 16 + 9
        n = jnp.where(c == 0, n0, n1)
        chunk_base = c * (16 * n0) + s * n
    else:
        n = ECHUNKS // 16
        chunk_base = s * n

    for p in range(n_phases):
        q = p * 2 + c
        table = table_ref if table_ref.ndim == 2 else table_ref.at[q]

        pltpu.sync_copy(zeros_ref.at[pl.ds(zbase, TILE_OUT)],
                        acc.at[pl.ds(zbase, TILE_OUT)])
        plsc.subcore_barrier()

        def idx_ds(i):
            rb = (chunk_base + i) * NSTR
            return [pltpu.make_async_copy(
                        src_ref.at[pl.ds(rb, NSTR)], ebs.at[i % 2], lsem),
                    pltpu.make_async_copy(
                        dst_ref.at[pl.ds(rb, NSTR)], ebd.at[i % 2], lsem)]

        def gather_ds(i):
            b = i % 2
            return [pltpu.make_async_copy(
                        table.at[ebs.at[b, j]], rowsv.at[b, j], gsem)
                    for j in range(NSTR)]

        def scatter_ds(i):
            b = i % 2
            return [pltpu.make_async_copy(
                        rowsv.at[b, j], acc.at[ebd.at[b, j]], ssem)
                    for j in range(NSTR)]

        def fire_scatters(i):
            b = i % 2
            for j in range(NSTR):
                pltpu.async_copy(rowsv.at[b, j], acc.at[ebd.at[b, j]],
                                 ssem, add=True)

        def sub(i, carry):
            for d in gather_ds(i):
                d.wait()
            for d in scatter_ds(i - 1):
                d.wait()
            for d in idx_ds(i + 1):
                d.start()
            fire_scatters(i)
            for d in idx_ds(i + 1):
                d.wait()
            for d in gather_ds(i + 1):
                d.start()
            return carry

        for d in idx_ds(0):
            d.start()
        for d in idx_ds(0):
            d.wait()
        for d in gather_ds(0):
            d.start()
        for d in idx_ds(1):
            d.start()
        for d in gather_ds(0):
            d.wait()
        fire_scatters(0)
        for d in idx_ds(1):
            d.wait()
        for d in gather_ds(1):
            d.start()
        lax.fori_loop(1, n - 1, sub, 0)
        for d in gather_ds(n - 1):
            d.wait()
        for d in scatter_ds(n - 2):
            d.wait()
        fire_scatters(n - 1)
        for d in scatter_ds(n - 1):
            d.wait()

        plsc.subcore_barrier()
        pltpu.sync_copy(acc.at[pl.ds(zbase, TILE_OUT)],
                        out_ref.at[q].at[pl.ds(zbase, TILE_OUT)])


def _make_sc_segsum(n_phases, split_edges):
    mesh = plsc.VectorSubcoreMesh(core_axis_name="c", subcore_axis_name="s")
    body = functools.partial(_sc_segsum, n_phases=n_phases,
                             split_edges=split_edges)
    return pl.kernel(
        body,
        out_type=jax.ShapeDtypeStruct((2 * n_phases, NPAD, FEAT),
                                      jnp.float32),
        mesh=mesh,
        scratch_types=[
            pltpu.VMEM((2, NSTR, SLEN), jnp.int32),
            pltpu.VMEM((2, NSTR, SLEN), jnp.int32),
            pltpu.VMEM((2, NSTR, SLEN, FEAT), jnp.float32),
            pltpu.VMEM_SHARED((NPAD, FEAT), jnp.float32),
            pltpu.SemaphoreType.DMA,
            pltpu.SemaphoreType.DMA,
            pltpu.SemaphoreType.DMA,
        ],
        compiler_params=pltpu.CompilerParams(use_tc_tiling_on_sc=False),
    )


def _tc_layer1(aggp_ref, x_ref, wrel_ref, wroot_ref, b_ref, out_ref):
    a = aggp_ref[0] + aggp_ref[1]
    x = x_ref[...]
    for m in range(8):
        am = a[:, m * FEAT:(m + 1) * FEAT]
        xm = x[:, m * FEAT:(m + 1) * FEAT]
        h = (jnp.dot(am, wrel_ref[...], preferred_element_type=jnp.float32)
             + jnp.dot(xm, wroot_ref[...], preferred_element_type=jnp.float32)
             + b_ref[...])
        h = jnp.maximum(h, 0.0)
        for qq in range(4):
            out_ref[qq, :, m * FEAT:(m + 1) * FEAT] = (
                h[:, qq * FEAT:(qq + 1) * FEAT])


def _tc_layer2(agg_ref, h_ref, wrel_ref, wroot_ref, b_ref, out_ref):
    rs = []
    for m in range(8):
        am = jnp.concatenate(
            [agg_ref[qq][:, m * FEAT:(m + 1) * FEAT] for qq in range(4)],
            axis=1)
        hm = jnp.concatenate(
            [h_ref[qq][:, m * FEAT:(m + 1) * FEAT] for qq in range(4)],
            axis=1)
        rs.append(
            jnp.dot(am, wrel_ref[...], preferred_element_type=jnp.float32)
            + jnp.dot(hm, wroot_ref[...], preferred_element_type=jnp.float32)
            + b_ref[...])
    blk = rs[0].shape[0]
    out_ref[...] = jnp.stack(rs, axis=1).reshape(blk * 8, SKEL)


_PBLK = 400
_GRID = PROWS // _PBLK


def kernel(x, edge_index, W1_rel, b1_rel, W1_root, W2_rel, b2_rel, W2_root):
    src = edge_index[0].astype(jnp.int32)
    dst = edge_index[1].astype(jnp.int32)
    npad_e = EPAD - N_EDGES
    srcp = jnp.concatenate([src, jnp.zeros((npad_e,), jnp.int32)])
    trash = N_NODES + (jnp.arange(npad_e, dtype=jnp.int32) % 1024)
    dstp = jnp.concatenate([dst, trash])
    src2 = srcp.reshape(ECHUNKS * NSTR, SLEN)
    dst2 = dstp.reshape(ECHUNKS * NSTR, SLEN)

    xv = jnp.pad(x, ((0, NPAD - N_NODES), (0, FEAT - 3)))
    xp = xv.reshape(PROWS, 128)
    w1rel = jnp.pad(W1_rel, ((0, FEAT - 3), (0, 0)))
    w1root = jnp.pad(W1_root, ((0, FEAT - 3), (0, 0)))
    zeros = jnp.zeros((NPAD, FEAT), jnp.float32)
    b1 = b1_rel.reshape(1, HIDDEN)
    b2 = b2_rel.reshape(1, SKEL)

    agg1p = _make_sc_segsum(n_phases=1, split_edges=True)(
        xv, src2, dst2, zeros)

    hq4 = pl.pallas_call(
        _tc_layer1,
        grid=(_GRID,),
        in_specs=[
            pl.BlockSpec((2, _PBLK, 128), lambda i: (0, i, 0)),
            pl.BlockSpec((_PBLK, 128), lambda i: (i, 0)),
            pl.BlockSpec((FEAT, HIDDEN), lambda i: (0, 0)),
            pl.BlockSpec((FEAT, HIDDEN), lambda i: (0, 0)),
            pl.BlockSpec((1, HIDDEN), lambda i: (0, 0)),
        ],
        out_specs=pl.BlockSpec((4, _PBLK, 128), lambda i: (0, i, 0)),
        out_shape=jax.ShapeDtypeStruct((4, PROWS, 128), jnp.float32),
    )(agg1p.reshape(2, PROWS, 128), xp, w1rel, w1root, b1)

    agg2q = _make_sc_segsum(n_phases=2, split_edges=False)(
        hq4.reshape(4, NPAD, FEAT), src2, dst2, zeros)

    out = pl.pallas_call(
        _tc_layer2,
        grid=(_GRID,),
        in_specs=[
            pl.BlockSpec((4, _PBLK, 128), lambda i: (0, i, 0)),
            pl.BlockSpec((4, _PBLK, 128), lambda i: (0, i, 0)),
            pl.BlockSpec((HIDDEN, SKEL), lambda i: (0, 0)),
            pl.BlockSpec((HIDDEN, SKEL), lambda i: (0, 0)),
            pl.BlockSpec((1, SKEL), lambda i: (0, 0)),
        ],
        out_specs=pl.BlockSpec((_PBLK * 8, SKEL), lambda i: (i, 0)),
        out_shape=jax.ShapeDtypeStruct((N_NODES, SKEL), jnp.float32),
    )(agg2q.reshape(4, PROWS, 128), hq4, W2_rel, W2_root, b2)
    return out


# speedup vs baseline: 1.2115x; 1.2115x over previous
import functools

import jax
import jax.numpy as jnp
from jax import lax
from jax.experimental import pallas as pl
from jax.experimental.pallas import tpu as pltpu
from jax.experimental.pallas import tpu_sc as plsc

N_NODES = 50000
HIDDEN = 64
SKEL = 256
FEAT = 16

NPAD = 51200
PROWS = NPAD * FEAT // 128
N_EDGES = 800000
SLEN = 384
NSTR = 1
EPAD = 811008
ECHUNKS = EPAD // (SLEN * NSTR)
TILE_OUT = NPAD // 16


def _sc_segsum(table_ref, src_ref, dst_ref, zeros_ref, out_ref,
               ebs, ebd, rowsv, acc, lsem, gsem, ssem,
               *, feat, split_edges):
    c = lax.axis_index("c")
    s = lax.axis_index("s")
    zbase = s * TILE_OUT

    if split_edges:
        n = (ECHUNKS // 2) // 16
        chunk_base = c * (ECHUNKS // 2) + s * n
    else:
        n = ECHUNKS // 16
        chunk_base = s * n

    for p in range(1):
        q = c
        table = table_ref if table_ref.ndim == 2 else table_ref.at[q]

        pltpu.sync_copy(zeros_ref.at[pl.ds(zbase, TILE_OUT)],
                        acc.at[pl.ds(zbase, TILE_OUT)])
        plsc.subcore_barrier()

        def idx_ds(i):
            rb = (chunk_base + i) * NSTR
            return [pltpu.make_async_copy(
                        src_ref.at[pl.ds(rb, NSTR)], ebs.at[i % 2], lsem),
                    pltpu.make_async_copy(
                        dst_ref.at[pl.ds(rb, NSTR)], ebd.at[i % 2], lsem)]

        def gather_ds(i):
            b = i % 2
            return [pltpu.make_async_copy(
                        table.at[ebs.at[b, j]], rowsv.at[b, j], gsem)
                    for j in range(NSTR)]

        def scatter_ds(i):
            b = i % 2
            return [pltpu.make_async_copy(
                        rowsv.at[b, j], acc.at[ebd.at[b, j]], ssem)
                    for j in range(NSTR)]

        def fire_scatters(i):
            b = i % 2
            for j in range(NSTR):
                pltpu.async_copy(rowsv.at[b, j], acc.at[ebd.at[b, j]],
                                 ssem, add=True)

        def sub(i, carry):
            for d in gather_ds(i):
                d.wait()
            for d in scatter_ds(i - 1):
                d.wait()
            for d in idx_ds(i + 1):
                d.start()
            fire_scatters(i)
            for d in idx_ds(i + 1):
                d.wait()
            for d in gather_ds(i + 1):
                d.start()
            return carry

        for d in idx_ds(0):
            d.start()
        for d in idx_ds(0):
            d.wait()
        for d in gather_ds(0):
            d.start()
        for d in idx_ds(1):
            d.start()
        for d in gather_ds(0):
            d.wait()
        fire_scatters(0)
        for d in idx_ds(1):
            d.wait()
        for d in gather_ds(1):
            d.start()
        lax.fori_loop(1, n - 1, sub, 0)
        for d in gather_ds(n - 1):
            d.wait()
        for d in scatter_ds(n - 2):
            d.wait()
        fire_scatters(n - 1)
        for d in scatter_ds(n - 1):
            d.wait()

        plsc.subcore_barrier()
        pltpu.sync_copy(acc.at[pl.ds(zbase, TILE_OUT)],
                        out_ref.at[q].at[pl.ds(zbase, TILE_OUT)])


def _make_sc_segsum(feat, split_edges):
    mesh = plsc.VectorSubcoreMesh(core_axis_name="c", subcore_axis_name="s")
    body = functools.partial(_sc_segsum, feat=feat, split_edges=split_edges)
    return pl.kernel(
        body,
        out_type=jax.ShapeDtypeStruct((2, NPAD, feat), jnp.float32),
        mesh=mesh,
        scratch_types=[
            pltpu.VMEM((2, NSTR, SLEN), jnp.int32),
            pltpu.VMEM((2, NSTR, SLEN), jnp.int32),
            pltpu.VMEM((2, NSTR, SLEN, feat), jnp.float32),
            pltpu.VMEM_SHARED((NPAD, feat), jnp.float32),
            pltpu.SemaphoreType.DMA,
            pltpu.SemaphoreType.DMA,
            pltpu.SemaphoreType.DMA,
        ],
        compiler_params=pltpu.CompilerParams(use_tc_tiling_on_sc=False),
    )


def _tc_layer1(aggp_ref, x_ref, wrel_ref, wroot_ref, b_ref, out_ref):
    a = aggp_ref[0] + aggp_ref[1]
    x = x_ref[...]
    for m in range(8):
        am = a[:, m * FEAT:(m + 1) * FEAT]
        xm = x[:, m * FEAT:(m + 1) * FEAT]
        h = (jnp.dot(am, wrel_ref[...], preferred_element_type=jnp.float32)
             + jnp.dot(xm, wroot_ref[...], preferred_element_type=jnp.float32)
             + b_ref[...])
        h = jnp.maximum(h, 0.0)
        for qq in range(2):
            out_ref[qq, :, m * 32:(m + 1) * 32] = h[:, qq * 32:(qq + 1) * 32]


def _tc_layer2(agg_ref, h_ref, wrel_ref, wroot_ref, b_ref, out_ref):
    rs = []
    for m in range(8):
        am = jnp.concatenate(
            [agg_ref[qq][:, m * 32:(m + 1) * 32] for qq in range(2)],
            axis=1)
        hm = jnp.concatenate(
            [h_ref[qq][:, m * 32:(m + 1) * 32] for qq in range(2)],
            axis=1)
        rs.append(
            jnp.dot(am, wrel_ref[...], preferred_element_type=jnp.float32)
            + jnp.dot(hm, wroot_ref[...], preferred_element_type=jnp.float32)
            + b_ref[...])
    blk = rs[0].shape[0]
    out_ref[...] = jnp.stack(rs, axis=1).reshape(blk * 8, SKEL)


_PBLK = 400
_GRID = PROWS // _PBLK


def kernel(x, edge_index, W1_rel, b1_rel, W1_root, W2_rel, b2_rel, W2_root):
    src = edge_index[0].astype(jnp.int32)
    dst = edge_index[1].astype(jnp.int32)
    npad_e = EPAD - N_EDGES
    srcp = jnp.concatenate([src, jnp.zeros((npad_e,), jnp.int32)])
    trash = N_NODES + (jnp.arange(npad_e, dtype=jnp.int32) % 1024)
    dstp = jnp.concatenate([dst, trash])
    src2 = srcp.reshape(ECHUNKS * NSTR, SLEN)
    dst2 = dstp.reshape(ECHUNKS * NSTR, SLEN)
    zeros32 = jnp.zeros((NPAD, 32), jnp.float32)

    xv = jnp.pad(x, ((0, NPAD - N_NODES), (0, FEAT - 3)))
    xp = xv.reshape(PROWS, 128)
    w1rel = jnp.pad(W1_rel, ((0, FEAT - 3), (0, 0)))
    w1root = jnp.pad(W1_root, ((0, FEAT - 3), (0, 0)))
    zeros = jnp.zeros((NPAD, FEAT), jnp.float32)
    b1 = b1_rel.reshape(1, HIDDEN)
    b2 = b2_rel.reshape(1, SKEL)

    agg1p = _make_sc_segsum(feat=FEAT, split_edges=True)(
        xv, src2, dst2, zeros)

    hq2 = pl.pallas_call(
        _tc_layer1,
        grid=(_GRID,),
        in_specs=[
            pl.BlockSpec((2, _PBLK, 128), lambda i: (0, i, 0)),
            pl.BlockSpec((_PBLK, 128), lambda i: (i, 0)),
            pl.BlockSpec((FEAT, HIDDEN), lambda i: (0, 0)),
            pl.BlockSpec((FEAT, HIDDEN), lambda i: (0, 0)),
            pl.BlockSpec((1, HIDDEN), lambda i: (0, 0)),
        ],
        out_specs=pl.BlockSpec((2, _PBLK, 256), lambda i: (0, i, 0)),
        out_shape=jax.ShapeDtypeStruct((2, PROWS, 256), jnp.float32),
    )(agg1p.reshape(2, PROWS, 128), xp, w1rel, w1root, b1)

    agg2q = _make_sc_segsum(feat=32, split_edges=False)(
        hq2.reshape(2, NPAD, 32), src2, dst2, zeros32)

    out = pl.pallas_call(
        _tc_layer2,
        grid=(_GRID,),
        in_specs=[
            pl.BlockSpec((2, _PBLK, 256), lambda i: (0, i, 0)),
            pl.BlockSpec((2, _PBLK, 256), lambda i: (0, i, 0)),
            pl.BlockSpec((HIDDEN, SKEL), lambda i: (0, 0)),
            pl.BlockSpec((HIDDEN, SKEL), lambda i: (0, 0)),
            pl.BlockSpec((1, SKEL), lambda i: (0, 0)),
        ],
        out_specs=pl.BlockSpec((_PBLK * 8, SKEL), lambda i: (i, 0)),
        out_shape=jax.ShapeDtypeStruct((N_NODES, SKEL), jnp.float32),
    )(agg2q.reshape(2, PROWS, 256), hq2, W2_rel, W2_root, b2)
    return out
